# baseline (speedup 1.0000x reference)
import functools

import jax
import jax.numpy as jnp
from jax import lax
from jax.experimental import pallas as pl
from jax.experimental.pallas import tpu as pltpu
from jax.experimental.pallas import tpu_sc as plsc

_N = 1_000_000
_TABLE = 128
_LANES = 16

_info = plsc.get_sparse_core_info()
_NC = _info.num_cores
_NS = _info.num_subcores
_NW = _NC * _NS

_UNROLL = 8
_PER_W = -(-_N // _NW)
_PER_W += (-_PER_W) % (_LANES * _UNROLL)
_NPAD = _PER_W * _NW
_NVEC = _PER_W // _LANES


@functools.partial(
    pl.kernel,
    out_type=jax.ShapeDtypeStruct((_NPAD,), jnp.float32),
    mesh=plsc.VectorSubcoreMesh(core_axis_name="c", subcore_axis_name="s"),
    compiler_params=pltpu.CompilerParams(needs_layout_passes=False),
    scratch_types=[
        pltpu.VMEM((_TABLE,), jnp.float32),
        pltpu.VMEM((_PER_W,), jnp.int32),
        pltpu.VMEM((_PER_W,), jnp.float32),
        pltpu.VMEM((_PER_W,), jnp.float32),
    ],
)
def _atomref_sc(x_hbm, z_hbm, tab_hbm, out_hbm, tab_v, z_v, x_v, o_v):
    wid = lax.axis_index("s") * _NC + lax.axis_index("c")
    base = wid * _PER_W

    pltpu.sync_copy(tab_hbm, tab_v)
    pltpu.sync_copy(z_hbm.at[pl.ds(base, _PER_W)], z_v)
    pltpu.sync_copy(x_hbm.at[pl.ds(base, _PER_W)], x_v)

    def body(i, _):
        off = i * (_LANES * _UNROLL)
        for j in range(_UNROLL):
            zv = z_v[pl.ds(off + j * _LANES, _LANES)]
            xv = x_v[pl.ds(off + j * _LANES, _LANES)]
            gv = plsc.load_gather(tab_v, [zv])
            o_v[pl.ds(off + j * _LANES, _LANES)] = xv + gv
        return 0

    lax.fori_loop(0, _NVEC // _UNROLL, body, 0)

    pltpu.sync_copy(o_v, out_hbm.at[pl.ds(base, _PER_W)])


def kernel(x, z, atomref_weight):
    xf = jnp.ravel(x).astype(jnp.float32)
    zi = jnp.ravel(z).astype(jnp.int32)
    tab = jnp.pad(jnp.ravel(atomref_weight).astype(jnp.float32),
                  (0, _TABLE - atomref_weight.shape[0]))
    xp = jnp.pad(xf, (0, _NPAD - _N))
    zp = jnp.pad(zi, (0, _NPAD - _N))
    out = _atomref_sc(xp, zp, tab)
    return out[:_N].reshape(_N, 1)


# speedup vs baseline: 1.0797x; 1.0797x over previous
import functools

import jax
import jax.numpy as jnp
from jax import lax
from jax.experimental import pallas as pl
from jax.experimental.pallas import tpu as pltpu
from jax.experimental.pallas import tpu_sc as plsc

_N = 1_000_000
_TABLE = 128
_LANES = 16

_info = plsc.get_sparse_core_info()
_NC = _info.num_cores
_NS = _info.num_subcores
_NW = _NC * _NS

_NVEC = _N // _LANES
_VPW = _NVEC // _NW
_PER_W = _VPW * _LANES
_REM_V = _NVEC - _VPW * _NW
_REM_BASE = _PER_W * _NW


@functools.partial(
    pl.kernel,
    out_type=jax.ShapeDtypeStruct((_N,), jnp.float32),
    mesh=plsc.VectorSubcoreMesh(core_axis_name="c", subcore_axis_name="s"),
    compiler_params=pltpu.CompilerParams(needs_layout_passes=False),
    scratch_types=[
        pltpu.VMEM((_TABLE,), jnp.float32),
        pltpu.VMEM((_PER_W + _LANES,), jnp.int32),
        pltpu.VMEM((_PER_W + _LANES,), jnp.float32),
        pltpu.VMEM((_PER_W + _LANES,), jnp.float32),
    ],
)
def _atomref_sc(x_hbm, z_hbm, tab_hbm, out_hbm, tab_v, z_v, x_v, o_v):
    wid = lax.axis_index("s") * _NC + lax.axis_index("c")
    base = wid * _PER_W

    pltpu.sync_copy(tab_hbm, tab_v)
    pltpu.sync_copy(z_hbm.at[pl.ds(base, _PER_W)], z_v.at[pl.ds(0, _PER_W)])
    pltpu.sync_copy(x_hbm.at[pl.ds(base, _PER_W)], x_v.at[pl.ds(0, _PER_W)])
    rem_off = _REM_BASE + wid * _LANES

    @pl.when(wid < _REM_V)
    def _load_extra():
        pltpu.sync_copy(z_hbm.at[pl.ds(rem_off, _LANES)],
                        z_v.at[pl.ds(_PER_W, _LANES)])
        pltpu.sync_copy(x_hbm.at[pl.ds(rem_off, _LANES)],
                        x_v.at[pl.ds(_PER_W, _LANES)])

    def body(i, _):
        off = i * _LANES
        zv = z_v[pl.ds(off, _LANES)]
        xv = x_v[pl.ds(off, _LANES)]
        gv = plsc.load_gather(tab_v, [zv])
        o_v[pl.ds(off, _LANES)] = xv + gv
        return 0

    lax.fori_loop(0, _VPW, body, 0)

    @pl.when(wid < _REM_V)
    def _do_extra():
        zv = z_v[pl.ds(_PER_W, _LANES)]
        xv = x_v[pl.ds(_PER_W, _LANES)]
        gv = plsc.load_gather(tab_v, [zv])
        o_v[pl.ds(_PER_W, _LANES)] = xv + gv

    pltpu.sync_copy(o_v.at[pl.ds(0, _PER_W)], out_hbm.at[pl.ds(base, _PER_W)])

    @pl.when(wid < _REM_V)
    def _store_extra():
        pltpu.sync_copy(o_v.at[pl.ds(_PER_W, _LANES)],
                        out_hbm.at[pl.ds(rem_off, _LANES)])


def kernel(x, z, atomref_weight):
    xf = jnp.ravel(x).astype(jnp.float32)
    zi = jnp.ravel(z).astype(jnp.int32)
    tab = jnp.pad(jnp.ravel(atomref_weight).astype(jnp.float32),
                  (0, _TABLE - atomref_weight.shape[0]))
    out = _atomref_sc(xf, zi, tab)
    return out.reshape(_N, 1)


# speedup vs baseline: 1.9688x; 1.8235x over previous
import functools

import jax
import jax.numpy as jnp
from jax import lax
from jax.experimental import pallas as pl
from jax.experimental.pallas import tpu as pltpu
from jax.experimental.pallas import tpu_sc as plsc

_N = 1_000_000
_TABLE = 128
_LANES = 16

_info = plsc.get_sparse_core_info()
_NC = _info.num_cores
_NS = _info.num_subcores
_NW = _NC * _NS

_NVEC = _N // _LANES
_VPW = _NVEC // _NW
_PER_W = _VPW * _LANES
_REM_V = _NVEC - _VPW * _NW
_REM_BASE = _PER_W * _NW


@functools.partial(
    pl.kernel,
    out_type=jax.ShapeDtypeStruct((_N,), jnp.float32),
    mesh=plsc.VectorSubcoreMesh(core_axis_name="c", subcore_axis_name="s"),
    compiler_params=pltpu.CompilerParams(needs_layout_passes=False),
    scratch_types=[
        pltpu.VMEM((_TABLE,), jnp.float32),
        pltpu.VMEM((_PER_W + _LANES,), jnp.int32),
        pltpu.VMEM((_PER_W + _LANES,), jnp.float32),
    ],
)
def _gather_sc(z_hbm, tab_hbm, out_hbm, tab_v, z_v, o_v):
    wid = lax.axis_index("s") * _NC + lax.axis_index("c")
    base = wid * _PER_W

    pltpu.sync_copy(tab_hbm, tab_v)
    pltpu.sync_copy(z_hbm.at[pl.ds(base, _PER_W)], z_v.at[pl.ds(0, _PER_W)])
    rem_off = _REM_BASE + wid * _LANES

    @pl.when(wid < _REM_V)
    def _load_extra():
        pltpu.sync_copy(z_hbm.at[pl.ds(rem_off, _LANES)],
                        z_v.at[pl.ds(_PER_W, _LANES)])

    def one_vec(off):
        zv = z_v[pl.ds(off, _LANES)]
        o_v[pl.ds(off, _LANES)] = plsc.load_gather(tab_v, [zv])

    def body(i, _):
        one_vec(i * _LANES)
        return 0

    lax.fori_loop(0, _VPW, body, 0)

    @pl.when(wid < _REM_V)
    def _do_extra():
        one_vec(_PER_W)

    pltpu.sync_copy(o_v.at[pl.ds(0, _PER_W)],
                    out_hbm.at[pl.ds(base, _PER_W)])

    @pl.when(wid < _REM_V)
    def _store_extra():
        pltpu.sync_copy(o_v.at[pl.ds(_PER_W, _LANES)],
                        out_hbm.at[pl.ds(rem_off, _LANES)])


def kernel(x, z, atomref_weight):
    zi = jnp.ravel(z).astype(jnp.int32)
    tab = jnp.pad(jnp.ravel(atomref_weight).astype(jnp.float32),
                  (0, _TABLE - atomref_weight.shape[0]))
    ref1d = _gather_sc(zi, tab)
    return x.astype(jnp.float32) + ref1d.reshape(_N, 1)


# speedup vs baseline: 2.4814x; 1.2604x over previous
import functools

import jax
import jax.numpy as jnp
from jax import lax
from jax.experimental import pallas as pl
from jax.experimental.pallas import tpu as pltpu
from jax.experimental.pallas import tpu_sc as plsc

_N = 1_000_000
_TABLE = 128
_LANES = 16

_info = plsc.get_sparse_core_info()
_NC = _info.num_cores
_NS = _info.num_subcores
_NW = _NC * _NS

_NVEC = _N // _LANES
_VPW = _NVEC // _NW
_PER_W = _VPW * _LANES
_REM_V = _NVEC - _VPW * _NW
_REM_BASE = _PER_W * _NW

_VA = 976
_VB = _VPW - _VA
_EA = _VA * _LANES
_EB = _VB * _LANES


@functools.partial(
    pl.kernel,
    out_type=jax.ShapeDtypeStruct((_N,), jnp.float32),
    mesh=plsc.VectorSubcoreMesh(core_axis_name="c", subcore_axis_name="s"),
    compiler_params=pltpu.CompilerParams(needs_layout_passes=False),
    scratch_types=[
        pltpu.VMEM((_TABLE,), jnp.float32),
        pltpu.VMEM((_PER_W + _LANES,), jnp.int32),
        pltpu.VMEM((_PER_W + _LANES,), jnp.float32),
        pltpu.SemaphoreType.DMA,
        pltpu.SemaphoreType.DMA,
        pltpu.SemaphoreType.DMA,
        pltpu.SemaphoreType.DMA,
    ],
)
def _gather_sc(z_hbm, tab_hbm, out_hbm, tab_v, z_v, o_v,
               sem_a, sem_b, sem_oa, sem_ob):
    wid = lax.axis_index("s") * _NC + lax.axis_index("c")
    base = wid * _PER_W
    rem_off = _REM_BASE + wid * _LANES

    cp_a = pltpu.async_copy(z_hbm.at[pl.ds(base, _EA)],
                            z_v.at[pl.ds(0, _EA)], sem_a)
    cp_b = pltpu.async_copy(z_hbm.at[pl.ds(base + _EA, _EB)],
                            z_v.at[pl.ds(_EA, _EB)], sem_b)
    pltpu.sync_copy(tab_hbm, tab_v)

    @pl.when(wid < _REM_V)
    def _load_extra():
        pltpu.sync_copy(z_hbm.at[pl.ds(rem_off, _LANES)],
                        z_v.at[pl.ds(_PER_W, _LANES)])

    def one_vec(off):
        zv = z_v[pl.ds(off, _LANES)]
        o_v[pl.ds(off, _LANES)] = plsc.load_gather(tab_v, [zv])

    cp_a.wait()

    @plsc.parallel_loop(0, _EA, _LANES, unroll=8)
    def _half_a(off):
        one_vec(off)

    out_a = pltpu.async_copy(o_v.at[pl.ds(0, _EA)],
                             out_hbm.at[pl.ds(base, _EA)], sem_oa)
    cp_b.wait()

    @plsc.parallel_loop(_EA, _EA + _EB - _LANES, _LANES, unroll=8)
    def _half_b(off):
        one_vec(off)

    one_vec(_EA + _EB - _LANES)

    @pl.when(wid < _REM_V)
    def _do_extra():
        one_vec(_PER_W)

    out_b = pltpu.async_copy(o_v.at[pl.ds(_EA, _EB)],
                             out_hbm.at[pl.ds(base + _EA, _EB)], sem_ob)

    @pl.when(wid < _REM_V)
    def _store_extra():
        pltpu.sync_copy(o_v.at[pl.ds(_PER_W, _LANES)],
                        out_hbm.at[pl.ds(rem_off, _LANES)])

    out_a.wait()
    out_b.wait()


def kernel(x, z, atomref_weight):
    zi = jnp.ravel(z).astype(jnp.int32)
    tab = jnp.pad(jnp.ravel(atomref_weight).astype(jnp.float32),
                  (0, _TABLE - atomref_weight.shape[0]))
    ref1d = _gather_sc(zi, tab)
    return x.astype(jnp.float32) + ref1d.reshape(_N, 1)
